# Initial kernel scaffold; baseline (speedup 1.0000x reference)
import jax
import jax.numpy as jnp
from jax.experimental import pallas as pl


def kernel(x_q, x_kv, q_w, q_dw_w, kv_w, kv_dw_w, proj_w, temperature, attn4):
    raise NotImplementedError("write your pallas kernel here")


# speedup vs baseline: 1.2546x; 1.2546x over previous
import jax
import jax.numpy as jnp
from jax import lax
from jax.experimental import pallas as pl

_TH = 16


def _conv_dw_path(xc, xt, xb, w1b, wd, t, nt, F, W_):
    f32 = jnp.float32
    bf = jnp.bfloat16
    y_c = jnp.dot(w1b, xc.astype(bf), preferred_element_type=f32)
    y_t = jnp.dot(w1b, xt.astype(bf), preferred_element_type=f32)
    y_b = jnp.dot(w1b, xb.astype(bf), preferred_element_type=f32)
    tmask = (t > 0).astype(f32)
    bmask = (t < nt - 1).astype(f32)
    ext = jnp.concatenate([y_t * tmask, y_c, y_b * bmask], axis=1)
    ext = ext.astype(bf).astype(f32)
    extp = jnp.pad(ext, ((0, 0), (1, 1)))
    jj = lax.broadcasted_iota(jnp.int32, (1, F), 1) % W_
    ml = (jj != (W_ - 1)).astype(f32)
    mr = (jj != 0).astype(f32)
    acc = jnp.zeros_like(xc)
    for di in range(3):
        for dj in range(3):
            term = extp[:, di * W_ + dj: di * W_ + dj + F]
            if dj == 0:
                term = term * mr
            elif dj == 2:
                term = term * ml
            acc = acc + wd[:, 3 * di + dj: 3 * di + dj + 1] * term
    return acc


def _stage_a1_body(F, W_):
    def body(xqc, xqt, xqb, xkc, xkt, xkb, qw, qdw, kw, kdw,
             qs_ref, ks_ref):
        t = pl.program_id(1)
        nt = pl.num_programs(1)
        q = _conv_dw_path(xqc[0], xqt[0], xqb[0], qw[...], qdw[...],
                          t, nt, F, W_)
        k = _conv_dw_path(xkc[0], xkt[0], xkb[0], kw[...], kdw[...],
                          t, nt, F, W_)
        qs = jnp.sum(q * q, axis=1, keepdims=True)
        ks = jnp.sum(k * k, axis=1, keepdims=True)

        @pl.when(t == 0)
        def _init():
            qs_ref[...] = jnp.zeros_like(qs_ref)
            ks_ref[...] = jnp.zeros_like(ks_ref)

        qs_ref[...] += qs[None]
        ks_ref[...] += ks[None]

    return body


def _stage_a2_body(heads, F, W_):
    def body(xqc, xqt, xqb, xkc, xkt, xkb, qw, qdw, kw, kdw,
             qs_ref, ks_ref, g_ref):
        t = pl.program_id(1)
        nt = pl.num_programs(1)
        bf = jnp.bfloat16
        q = _conv_dw_path(xqc[0], xqt[0], xqb[0], qw[...], qdw[...],
                          t, nt, F, W_)
        k = _conv_dw_path(xkc[0], xkt[0], xkb[0], kw[...], kdw[...],
                          t, nt, F, W_)
        C = q.shape[0]
        Ch = C // heads
        sq = jnp.maximum(jnp.sqrt(qs_ref[0]), 1e-12)
        sk = jnp.maximum(jnp.sqrt(ks_ref[0]), 1e-12)
        qb = (q / sq).astype(bf)
        kb = (k / sk).astype(bf)
        g_parts = []
        for h in range(heads):
            g_parts.append(lax.dot_general(
                qb[h * Ch:(h + 1) * Ch], kb[h * Ch:(h + 1) * Ch],
                (((1,), (1,)), ((), ())),
                preferred_element_type=jnp.float32))
        g96 = jnp.concatenate(g_parts, axis=0)

        @pl.when(t == 0)
        def _init():
            g_ref[...] = jnp.zeros_like(g_ref)

        g_ref[...] += g96[None]

    return body


def _stage_b_body(heads):
    def body(g_ref, temp_ref, s_ref):
        BIG = jnp.float32(3.0e38)
        attn = g_ref[0] * temp_ref[...]
        C, Ch = attn.shape
        ndrop = Ch - int(Ch * 0.9)
        colid = lax.broadcasted_iota(jnp.int32, (C, Ch), 1)
        A = attn
        dropped = jnp.zeros((C, Ch), jnp.bool_)
        for _ in range(ndrop):
            m = jnp.min(A, axis=1, keepdims=True)
            cand = jnp.where(A == m, colid, -1)
            jmax = jnp.max(cand, axis=1, keepdims=True)
            drop = colid == jmax
            dropped = jnp.logical_or(dropped, drop)
            A = jnp.where(drop, BIG, A)
        kept = jnp.logical_not(dropped)
        mx = jnp.max(jnp.where(kept, attn, -BIG), axis=1, keepdims=True)
        e = jnp.where(kept, jnp.exp(attn - mx), 0.0)
        s_ref[...] = (e / jnp.sum(e, axis=1, keepdims=True))[None]

    return body


def _stage_c_body(heads, F, W_):
    def body(xkc, xkt, xkb, vw, vdw, s_ref, projb_ref, a4_ref, out_ref):
        t = pl.program_id(1)
        nt = pl.num_programs(1)
        f32 = jnp.float32
        bf = jnp.bfloat16
        v = _conv_dw_path(xkc[0], xkt[0], xkb[0], vw[...], vdw[...],
                          t, nt, F, W_)
        C = v.shape[0]
        Ch = C // heads
        vb = v.astype(bf)
        sb = s_ref[0].astype(bf)
        parts = []
        for h in range(heads):
            parts.append(jnp.dot(sb[h * Ch:(h + 1) * Ch],
                                 vb[h * Ch:(h + 1) * Ch],
                                 preferred_element_type=f32))
        out1 = jnp.concatenate(parts, axis=0) * a4_ref[0, 0]
        out_ref[...] = jnp.dot(projb_ref[...], out1.astype(bf),
                               preferred_element_type=f32)[None]

    return body


def kernel(x_q, x_kv, q_w, q_dw_w, kv_w, kv_dw_w, proj_w, temperature, attn4):
    B_, C, H_, W_ = x_q.shape
    heads = temperature.shape[0]
    Ch = C // heads
    N_ = H_ * W_
    Th = _TH
    nT = H_ // Th
    F = Th * W_
    f32 = jnp.float32
    bf = jnp.bfloat16

    xq2 = x_q.reshape(B_, C, N_)
    xk2 = x_kv.reshape(B_, C, N_)
    qwb = q_w.reshape(C, C).astype(bf)
    kwb = kv_w[:C].reshape(C, C).astype(bf)
    vwb = kv_w[C:].reshape(C, C).astype(bf)
    qdw2 = q_dw_w.reshape(C, 9)
    kdw2 = kv_dw_w[:C].reshape(C, 9)
    vdw2 = kv_dw_w[C:].reshape(C, 9)
    projb = proj_w.reshape(C, C).astype(bf)
    temp_rows = jnp.repeat(temperature.reshape(heads), Ch).reshape(C, 1)
    a4 = attn4.reshape(1, 1)

    center = pl.BlockSpec((1, C, F), lambda b, t: (b, 0, t))
    halo_t = pl.BlockSpec((1, C, W_),
                          lambda b, t: (b, 0, jnp.maximum(t * Th - 1, 0)))
    halo_b = pl.BlockSpec((1, C, W_),
                          lambda b, t: (b, 0, jnp.minimum(t * Th + Th, H_ - 1)))
    wfull = lambda shape: pl.BlockSpec(shape, lambda b, t: (0,) * len(shape))
    acc_spec = lambda shape: pl.BlockSpec(shape, lambda b, t: (b,) + (0,) * (len(shape) - 1))

    conv_ins = [center, halo_t, halo_b, center, halo_t, halo_b,
                wfull((C, C)), wfull((C, 9)), wfull((C, C)), wfull((C, 9))]

    qs, ks = pl.pallas_call(
        _stage_a1_body(F, W_),
        grid=(B_, nT),
        in_specs=conv_ins,
        out_specs=[acc_spec((1, C, 1)), acc_spec((1, C, 1))],
        out_shape=[jax.ShapeDtypeStruct((B_, C, 1), f32),
                   jax.ShapeDtypeStruct((B_, C, 1), f32)],
    )(xq2, xq2, xq2, xk2, xk2, xk2, qwb, qdw2, kwb, kdw2)

    g = pl.pallas_call(
        _stage_a2_body(heads, F, W_),
        grid=(B_, nT),
        in_specs=conv_ins + [acc_spec((1, C, 1)), acc_spec((1, C, 1))],
        out_specs=acc_spec((1, C, Ch)),
        out_shape=jax.ShapeDtypeStruct((B_, C, Ch), f32),
    )(xq2, xq2, xq2, xk2, xk2, xk2, qwb, qdw2, kwb, kdw2, qs, ks)

    s = pl.pallas_call(
        _stage_b_body(heads),
        grid=(B_,),
        in_specs=[pl.BlockSpec((1, C, Ch), lambda b: (b, 0, 0)),
                  pl.BlockSpec((C, 1), lambda b: (0, 0))],
        out_specs=pl.BlockSpec((1, C, Ch), lambda b: (b, 0, 0)),
        out_shape=jax.ShapeDtypeStruct((B_, C, Ch), f32),
    )(g, temp_rows)

    out = pl.pallas_call(
        _stage_c_body(heads, F, W_),
        grid=(B_, nT),
        in_specs=[center, halo_t, halo_b, wfull((C, C)), wfull((C, 9)),
                  acc_spec((1, C, Ch)), wfull((C, C)), wfull((1, 1))],
        out_specs=pl.BlockSpec((1, C, F), lambda b, t: (b, 0, t)),
        out_shape=jax.ShapeDtypeStruct((B_, C, N_), f32),
    )(xk2, xk2, xk2, vwb, vdw2, s, projb, a4)

    return out.reshape(B_, C, H_, W_)


# speedup vs baseline: 1.9965x; 1.5913x over previous
import jax
import jax.numpy as jnp
from jax import lax
from jax.experimental import pallas as pl
from jax.experimental.pallas import tpu as pltpu

_TH = 16


def _conv_dw_path(xc, xt, xb, w1b, wd, t, nt, F, W_):
    f32 = jnp.float32
    bf = jnp.bfloat16
    y_c = jnp.dot(w1b, xc.astype(bf), preferred_element_type=f32)
    y_t = jnp.dot(w1b, xt.astype(bf), preferred_element_type=f32)
    y_b = jnp.dot(w1b, xb.astype(bf), preferred_element_type=f32)
    tmask = (t > 0).astype(f32)
    bmask = (t < nt - 1).astype(f32)
    ext = jnp.concatenate([y_t * tmask, y_c, y_b * bmask], axis=1)
    ext = ext.astype(bf).astype(f32)
    E = F + 2 * W_
    zcol = jnp.zeros((ext.shape[0], 1), f32)
    jj = lax.broadcasted_iota(jnp.int32, (1, E), 1) % W_
    left = jnp.concatenate([zcol, ext[:, :-1]], axis=1) * (jj != 0).astype(f32)
    right = jnp.concatenate([ext[:, 1:], zcol], axis=1) * \
        (jj != (W_ - 1)).astype(f32)
    srcs = (left, ext, right)
    acc = jnp.zeros_like(xc)
    for di in range(3):
        for dj in range(3):
            acc = acc + (wd[:, 3 * di + dj: 3 * di + dj + 1] *
                         srcs[dj][:, di * W_: di * W_ + F])
    return acc


def _stage_a1_body(F, W_):
    def body(xqc, xqt, xqb, xkc, xkt, xkb, qw, qdw, kw, kdw,
             qs_ref, ks_ref):
        t = pl.program_id(1)
        nt = pl.num_programs(1)
        q = _conv_dw_path(xqc[0], xqt[0], xqb[0], qw[...], qdw[...],
                          t, nt, F, W_)
        k = _conv_dw_path(xkc[0], xkt[0], xkb[0], kw[...], kdw[...],
                          t, nt, F, W_)
        qs = jnp.sum(q * q, axis=1, keepdims=True)
        ks = jnp.sum(k * k, axis=1, keepdims=True)

        @pl.when(t == 0)
        def _init():
            qs_ref[...] = jnp.zeros_like(qs_ref)
            ks_ref[...] = jnp.zeros_like(ks_ref)

        qs_ref[...] += qs[None]
        ks_ref[...] += ks[None]

    return body


def _stage_a2_body(heads, F, W_):
    def body(xqc, xqt, xqb, xkc, xkt, xkb, qw, qdw, kw, kdw,
             qs_ref, ks_ref, g_ref):
        t = pl.program_id(1)
        nt = pl.num_programs(1)
        bf = jnp.bfloat16
        q = _conv_dw_path(xqc[0], xqt[0], xqb[0], qw[...], qdw[...],
                          t, nt, F, W_)
        k = _conv_dw_path(xkc[0], xkt[0], xkb[0], kw[...], kdw[...],
                          t, nt, F, W_)
        C = q.shape[0]
        Ch = C // heads
        sq = jnp.maximum(jnp.sqrt(qs_ref[0]), 1e-12)
        sk = jnp.maximum(jnp.sqrt(ks_ref[0]), 1e-12)
        qb = (q / sq).astype(bf)
        kb = (k / sk).astype(bf)
        g_parts = []
        for h in range(heads):
            g_parts.append(lax.dot_general(
                qb[h * Ch:(h + 1) * Ch], kb[h * Ch:(h + 1) * Ch],
                (((1,), (1,)), ((), ())),
                preferred_element_type=jnp.float32))
        g96 = jnp.concatenate(g_parts, axis=0)

        @pl.when(t == 0)
        def _init():
            g_ref[...] = jnp.zeros_like(g_ref)

        g_ref[...] += g96[None]

    return body


def _stage_b_body(heads):
    def body(g_ref, temp_ref, s_ref):
        BIG = jnp.float32(3.0e38)
        attn = g_ref[0] * temp_ref[...]
        C, Ch = attn.shape
        ndrop = Ch - int(Ch * 0.9)
        colid = lax.broadcasted_iota(jnp.int32, (C, Ch), 1)
        A = attn
        dropped = jnp.zeros((C, Ch), jnp.bool_)
        for _ in range(ndrop):
            m = jnp.min(A, axis=1, keepdims=True)
            cand = jnp.where(A == m, colid, -1)
            jmax = jnp.max(cand, axis=1, keepdims=True)
            drop = colid == jmax
            dropped = jnp.logical_or(dropped, drop)
            A = jnp.where(drop, BIG, A)
        kept = jnp.logical_not(dropped)
        mx = jnp.max(jnp.where(kept, attn, -BIG), axis=1, keepdims=True)
        e = jnp.where(kept, jnp.exp(attn - mx), 0.0)
        s_ref[...] = (e / jnp.sum(e, axis=1, keepdims=True))[None]

    return body


def _stage_c_body(heads, F, W_):
    def body(xkc, xkt, xkb, vw, vdw, s_ref, projb_ref, a4_ref, out_ref):
        t = pl.program_id(1)
        nt = pl.num_programs(1)
        f32 = jnp.float32
        bf = jnp.bfloat16
        v = _conv_dw_path(xkc[0], xkt[0], xkb[0], vw[...], vdw[...],
                          t, nt, F, W_)
        C = v.shape[0]
        Ch = C // heads
        vb = v.astype(bf)
        sb = s_ref[0].astype(bf)
        parts = []
        for h in range(heads):
            parts.append(jnp.dot(sb[h * Ch:(h + 1) * Ch],
                                 vb[h * Ch:(h + 1) * Ch],
                                 preferred_element_type=f32))
        out1 = jnp.concatenate(parts, axis=0) * a4_ref[0, 0]
        out_ref[...] = jnp.dot(projb_ref[...], out1.astype(bf),
                               preferred_element_type=f32)[None]

    return body


def kernel(x_q, x_kv, q_w, q_dw_w, kv_w, kv_dw_w, proj_w, temperature, attn4):
    B_, C, H_, W_ = x_q.shape
    heads = temperature.shape[0]
    Ch = C // heads
    N_ = H_ * W_
    Th = _TH
    nT = H_ // Th
    F = Th * W_
    f32 = jnp.float32
    bf = jnp.bfloat16

    xq2 = x_q.reshape(B_, C, N_)
    xk2 = x_kv.reshape(B_, C, N_)
    qwb = q_w.reshape(C, C).astype(bf)
    kwb = kv_w[:C].reshape(C, C).astype(bf)
    vwb = kv_w[C:].reshape(C, C).astype(bf)
    qdw2 = q_dw_w.reshape(C, 9)
    kdw2 = kv_dw_w[:C].reshape(C, 9)
    vdw2 = kv_dw_w[C:].reshape(C, 9)
    projb = proj_w.reshape(C, C).astype(bf)
    temp_rows = jnp.repeat(temperature.reshape(heads), Ch).reshape(C, 1)
    a4 = attn4.reshape(1, 1)

    center = pl.BlockSpec((1, C, F), lambda b, t: (b, 0, t))
    halo_t = pl.BlockSpec((1, C, W_),
                          lambda b, t: (b, 0, jnp.maximum(t * Th - 1, 0)))
    halo_b = pl.BlockSpec((1, C, W_),
                          lambda b, t: (b, 0, jnp.minimum(t * Th + Th, H_ - 1)))
    wfull = lambda shape: pl.BlockSpec(shape, lambda b, t: (0,) * len(shape))
    acc_spec = lambda shape: pl.BlockSpec(shape, lambda b, t: (b,) + (0,) * (len(shape) - 1))

    conv_ins = [center, halo_t, halo_b, center, halo_t, halo_b,
                wfull((C, C)), wfull((C, 9)), wfull((C, C)), wfull((C, 9))]

    qs, ks = pl.pallas_call(
        _stage_a1_body(F, W_),
        grid=(B_, nT),
        in_specs=conv_ins,
        out_specs=[acc_spec((1, C, 1)), acc_spec((1, C, 1))],
        out_shape=[jax.ShapeDtypeStruct((B_, C, 1), f32),
                   jax.ShapeDtypeStruct((B_, C, 1), f32)],
        compiler_params=pltpu.CompilerParams(
            dimension_semantics=("parallel", "arbitrary")),
    )(xq2, xq2, xq2, xk2, xk2, xk2, qwb, qdw2, kwb, kdw2)

    g = pl.pallas_call(
        _stage_a2_body(heads, F, W_),
        grid=(B_, nT),
        in_specs=conv_ins + [acc_spec((1, C, 1)), acc_spec((1, C, 1))],
        out_specs=acc_spec((1, C, Ch)),
        out_shape=jax.ShapeDtypeStruct((B_, C, Ch), f32),
        compiler_params=pltpu.CompilerParams(
            dimension_semantics=("parallel", "arbitrary")),
    )(xq2, xq2, xq2, xk2, xk2, xk2, qwb, qdw2, kwb, kdw2, qs, ks)

    s = pl.pallas_call(
        _stage_b_body(heads),
        grid=(B_,),
        in_specs=[pl.BlockSpec((1, C, Ch), lambda b: (b, 0, 0)),
                  pl.BlockSpec((C, 1), lambda b: (0, 0))],
        out_specs=pl.BlockSpec((1, C, Ch), lambda b: (b, 0, 0)),
        out_shape=jax.ShapeDtypeStruct((B_, C, Ch), f32),
    )(g, temp_rows)

    out = pl.pallas_call(
        _stage_c_body(heads, F, W_),
        grid=(B_, nT),
        in_specs=[center, halo_t, halo_b, wfull((C, C)), wfull((C, 9)),
                  acc_spec((1, C, Ch)), wfull((C, C)), wfull((1, 1))],
        out_specs=pl.BlockSpec((1, C, F), lambda b, t: (b, 0, t)),
        out_shape=jax.ShapeDtypeStruct((B_, C, N_), f32),
        compiler_params=pltpu.CompilerParams(
            dimension_semantics=("parallel", "parallel")),
    )(xk2, xk2, xk2, vwb, vdw2, s, projb, a4)

    return out.reshape(B_, C, H_, W_)
